# Initial kernel scaffold; baseline (speedup 1.0000x reference)
import jax
import jax.numpy as jnp
from jax.experimental import pallas as pl


def kernel(edge_index, confidences, num_nodes):
    raise NotImplementedError("write your pallas kernel here")


# speedup vs baseline: 347.5928x; 347.5928x over previous
import functools

import jax
import jax.numpy as jnp
from jax import lax
from jax.experimental import pallas as pl
from jax.experimental.pallas import tpu as pltpu
from jax.experimental.pallas import tpu_sc as plsc

NUM_CORES = 2
NUM_SUBCORES = 16
LANES = 16
NW = NUM_CORES * NUM_SUBCORES


def _pick_chunk(e_per_w: int) -> int:
    for c in range(min(e_per_w, 8192), 0, -16):
        if e_per_w % c == 0:
            return c
    return e_per_w


@functools.lru_cache(maxsize=None)
def _make_sc_kernel(n_edges: int, n_nodes: int):
    assert n_edges % (NW * LANES) == 0
    e_per_w = n_edges // NW
    chunk = _pick_chunk(e_per_w)
    n_chunks = e_per_w // chunk
    mesh = plsc.VectorSubcoreMesh(
        core_axis_name="c", subcore_axis_name="s",
        num_cores=NUM_CORES, num_subcores=NUM_SUBCORES)

    @functools.partial(
        pl.kernel,
        mesh=mesh,
        out_type=jax.ShapeDtypeStruct((n_edges,), jnp.float32),
        scratch_types=[
            pltpu.VMEM((n_nodes,), jnp.float32),
            pltpu.VMEM((chunk,), jnp.int32),
            pltpu.VMEM((chunk,), jnp.int32),
            pltpu.VMEM((chunk,), jnp.float32),
        ],
        compiler_params=pltpu.CompilerParams(needs_layout_passes=False),
    )
    def k(ei_hbm, conf_hbm, out_hbm, conf_v, src_v, dst_v, w_v):
        wid = lax.axis_index("s") * NUM_CORES + lax.axis_index("c")
        pltpu.sync_copy(conf_hbm, conf_v)
        base = wid * e_per_w

        def chunk_body(ci, carry):
            cbase = base + ci * chunk
            pltpu.sync_copy(ei_hbm.at[pl.ds(cbase, chunk)], src_v)
            pltpu.sync_copy(ei_hbm.at[pl.ds(n_edges + cbase, chunk)], dst_v)

            def vec_body(i, c2):
                o = i * LANES
                si = src_v[pl.ds(o, LANES)]
                di = dst_v[pl.ds(o, LANES)]
                cs = plsc.load_gather(conf_v, [si])
                cd = plsc.load_gather(conf_v, [di])
                w_v[pl.ds(o, LANES)] = jnp.exp(-jnp.abs(cs - cd))
                return c2

            lax.fori_loop(0, chunk // LANES, vec_body, 0)
            pltpu.sync_copy(w_v, out_hbm.at[pl.ds(cbase, chunk)])
            return carry

        lax.fori_loop(0, n_chunks, chunk_body, 0)

    return k


def kernel(edge_index, confidences, num_nodes):
    del num_nodes
    n_edges = edge_index.shape[1]
    ei_flat = edge_index.reshape(-1)
    w = _make_sc_kernel(n_edges, confidences.shape[0])(ei_flat, confidences)
    return (edge_index, w)


# speedup vs baseline: 627.4219x; 1.8050x over previous
import functools

import jax
import jax.numpy as jnp
from jax import lax
from jax.experimental import pallas as pl
from jax.experimental.pallas import tpu as pltpu
from jax.experimental.pallas import tpu_sc as plsc

NUM_CORES = 2
NUM_SUBCORES = 16
LANES = 16
NW = NUM_CORES * NUM_SUBCORES


def _pick_chunk(e_per_w: int) -> int:
    for c in range(min(e_per_w, 8192), 0, -16):
        if e_per_w % c == 0:
            return c
    return e_per_w


@functools.lru_cache(maxsize=None)
def _make_sc_kernel(n_edges: int, n_nodes: int):
    assert n_edges % (NW * LANES) == 0
    e_per_w = n_edges // NW
    chunk = _pick_chunk(e_per_w)
    n_chunks = e_per_w // chunk
    mesh = plsc.VectorSubcoreMesh(
        core_axis_name="c", subcore_axis_name="s",
        num_cores=NUM_CORES, num_subcores=NUM_SUBCORES)

    @functools.partial(
        pl.kernel,
        mesh=mesh,
        out_type=jax.ShapeDtypeStruct((n_edges,), jnp.float32),
        scratch_types=[
            pltpu.VMEM((n_nodes,), jnp.float32),
            pltpu.VMEM((chunk,), jnp.int32),
            pltpu.VMEM((chunk,), jnp.int32),
            pltpu.VMEM((chunk,), jnp.float32),
        ],
        compiler_params=pltpu.CompilerParams(needs_layout_passes=False),
    )
    def k(ei_hbm, conf_hbm, out_hbm, conf_v, src_v, dst_v, w_v):
        wid = lax.axis_index("s") * NUM_CORES + lax.axis_index("c")
        pltpu.sync_copy(conf_hbm, conf_v)
        base = wid * e_per_w

        def chunk_body(ci, carry):
            cbase = base + ci * chunk
            pltpu.sync_copy(ei_hbm.at[pl.ds(cbase, chunk)], src_v)
            pltpu.sync_copy(ei_hbm.at[pl.ds(n_edges + cbase, chunk)], dst_v)

            @plsc.parallel_loop(0, chunk, LANES, unroll=8)
            def vec_body(o):
                si = src_v[pl.ds(o, LANES)]
                di = dst_v[pl.ds(o, LANES)]
                cs = plsc.load_gather(conf_v, [si])
                cd = plsc.load_gather(conf_v, [di])
                w_v[pl.ds(o, LANES)] = jnp.exp(-jnp.abs(cs - cd))
            pltpu.sync_copy(w_v, out_hbm.at[pl.ds(cbase, chunk)])
            return carry

        lax.fori_loop(0, n_chunks, chunk_body, 0)

    return k


def kernel(edge_index, confidences, num_nodes):
    del num_nodes
    n_edges = edge_index.shape[1]
    ei_flat = edge_index.reshape(-1)
    w = _make_sc_kernel(n_edges, confidences.shape[0])(ei_flat, confidences)
    return (edge_index, w)


# speedup vs baseline: 819.3380x; 1.3059x over previous
import functools

import jax
import jax.numpy as jnp
from jax import lax
from jax.experimental import pallas as pl
from jax.experimental.pallas import tpu as pltpu
from jax.experimental.pallas import tpu_sc as plsc

NUM_CORES = 2
NUM_SUBCORES = 16
LANES = 16
NW = NUM_CORES * NUM_SUBCORES
NBUF = 2


def _pick_chunk(e_per_w: int) -> int:
    for c in range(min(e_per_w // NBUF, 4096), 0, -16):
        if e_per_w % (NBUF * c) == 0:
            return c
    return e_per_w // NBUF


@functools.lru_cache(maxsize=None)
def _make_sc_kernel(n_edges: int, n_nodes: int):
    assert n_edges % (NW * LANES) == 0
    e_per_w = n_edges // NW
    chunk = _pick_chunk(e_per_w)
    n_chunks = e_per_w // chunk
    mesh = plsc.VectorSubcoreMesh(
        core_axis_name="c", subcore_axis_name="s",
        num_cores=NUM_CORES, num_subcores=NUM_SUBCORES)

    @functools.partial(
        pl.kernel,
        mesh=mesh,
        out_type=jax.ShapeDtypeStruct((n_edges,), jnp.float32),
        scratch_types=[
            pltpu.VMEM((n_nodes,), jnp.float32),
            [pltpu.VMEM((chunk,), jnp.int32)] * NBUF,
            [pltpu.VMEM((chunk,), jnp.int32)] * NBUF,
            [pltpu.VMEM((chunk,), jnp.float32)] * NBUF,
            [pltpu.SemaphoreType.DMA] * NBUF,
            [pltpu.SemaphoreType.DMA] * NBUF,
        ],
        compiler_params=pltpu.CompilerParams(needs_layout_passes=False),
    )
    def k(ei_hbm, conf_hbm, out_hbm, conf_v, src_v, dst_v, w_v, sin, sout):
        wid = lax.axis_index("s") * NUM_CORES + lax.axis_index("c")
        base = wid * e_per_w
        pltpu.sync_copy(conf_hbm, conf_v)

        def start_in(ci, b):
            cbase = base + ci * chunk
            pltpu.async_copy(ei_hbm.at[pl.ds(cbase, chunk)],
                             src_v[b], sin[b])
            pltpu.async_copy(ei_hbm.at[pl.ds(n_edges + cbase, chunk)],
                             dst_v[b], sin[b])

        def wait_in(ci, b):
            cbase = base + ci * chunk
            pltpu.make_async_copy(ei_hbm.at[pl.ds(cbase, chunk)],
                                  src_v[b], sin[b]).wait()
            pltpu.make_async_copy(ei_hbm.at[pl.ds(n_edges + cbase, chunk)],
                                  dst_v[b], sin[b]).wait()

        def start_out(ci, b):
            cbase = base + ci * chunk
            pltpu.async_copy(w_v[b], out_hbm.at[pl.ds(cbase, chunk)],
                             sout[b])

        def wait_out(ci, b):
            cbase = base + ci * chunk
            pltpu.make_async_copy(w_v[b],
                                  out_hbm.at[pl.ds(cbase, chunk)],
                                  sout[b]).wait()

        for b in range(NBUF):
            start_in(b, b)

        def outer(g, carry):
            for b in range(NBUF):
                ci = g * NBUF + b
                wait_in(ci, b)

                @pl.when(ci >= NBUF)
                def _():
                    wait_out(ci - NBUF, b)

                @plsc.parallel_loop(0, chunk, LANES, unroll=8)
                def vec_body(o):
                    si = src_v[b][pl.ds(o, LANES)]
                    di = dst_v[b][pl.ds(o, LANES)]
                    cs = plsc.load_gather(conf_v, [si])
                    cd = plsc.load_gather(conf_v, [di])
                    w_v[b][pl.ds(o, LANES)] = jnp.exp(-jnp.abs(cs - cd))

                start_out(ci, b)

                @pl.when(ci + NBUF < n_chunks)
                def _():
                    start_in(ci + NBUF, b)
            return carry

        lax.fori_loop(0, n_chunks // NBUF, outer, 0)
        for b in range(NBUF):
            wait_out(n_chunks - NBUF + b, b)

    return k


def kernel(edge_index, confidences, num_nodes):
    del num_nodes
    n_edges = edge_index.shape[1]
    ei_flat = edge_index.reshape(-1)
    w = _make_sc_kernel(n_edges, confidences.shape[0])(ei_flat, confidences)
    return (edge_index, w)


# speedup vs baseline: 972.3321x; 1.1867x over previous
import functools

import jax
import jax.numpy as jnp
from jax import lax
from jax.experimental import pallas as pl
from jax.experimental.pallas import tpu as pltpu
from jax.experimental.pallas import tpu_sc as plsc

NUM_CORES = 2
NUM_SUBCORES = 16
LANES = 16
NW = NUM_CORES * NUM_SUBCORES
NBUF = 2
BLK = 128
CHUNK = 2048


@functools.lru_cache(maxsize=None)
def _make_sc_kernel(n_edges: int, n_nodes: int):
    assert n_edges % CHUNK == 0
    n_chunks = n_edges // CHUNK
    trips = -(-n_chunks // NW)
    mesh = plsc.VectorSubcoreMesh(
        core_axis_name="c", subcore_axis_name="s",
        num_cores=NUM_CORES, num_subcores=NUM_SUBCORES)

    @functools.partial(
        pl.kernel,
        mesh=mesh,
        out_type=jax.ShapeDtypeStruct((n_edges,), jnp.float32),
        scratch_types=[
            pltpu.VMEM((n_nodes,), jnp.float32),
            [pltpu.VMEM((2, CHUNK), jnp.int32)] * NBUF,
            [pltpu.VMEM((CHUNK,), jnp.float32)] * NBUF,
            [pltpu.SemaphoreType.DMA] * NBUF,
            [pltpu.SemaphoreType.DMA] * NBUF,
        ],
        compiler_params=pltpu.CompilerParams(needs_layout_passes=False),
    )
    def k(ei_hbm, conf_hbm, out_hbm, conf_v, ei_v, w_v, sin, sout):
        wid = lax.axis_index("s") * NUM_CORES + lax.axis_index("c")
        pltpu.sync_copy(conf_hbm, conf_v)

        def cbase_of(ci):
            return jnp.minimum(wid + ci * NW, n_chunks - 1) * CHUNK

        def start_in(ci, b):
            ebase = cbase_of(ci)
            pltpu.async_copy(ei_hbm.at[:, pl.ds(ebase, CHUNK)],
                             ei_v[b], sin[b])

        def wait_in(ci, b):
            ebase = cbase_of(ci)
            pltpu.make_async_copy(ei_hbm.at[:, pl.ds(ebase, CHUNK)],
                                  ei_v[b], sin[b]).wait()

        def start_out(ci, b):
            ebase = cbase_of(ci)
            pltpu.async_copy(w_v[b], out_hbm.at[pl.ds(ebase, CHUNK)],
                             sout[b])

        def wait_out(ci, b):
            ebase = cbase_of(ci)
            pltpu.make_async_copy(w_v[b],
                                  out_hbm.at[pl.ds(ebase, CHUNK)],
                                  sout[b]).wait()

        for b in range(NBUF):
            start_in(b, b)

        def outer(g, carry):
            for b in range(NBUF):
                ci = g * NBUF + b
                wait_in(ci, b)

                @pl.when(ci >= NBUF)
                def _():
                    wait_out(ci - NBUF, b)

                @plsc.parallel_loop(0, CHUNK, LANES, unroll=8)
                def vec_body(o):
                    si = ei_v[b][0, pl.ds(o, LANES)]
                    di = ei_v[b][1, pl.ds(o, LANES)]
                    cs = plsc.load_gather(conf_v, [si])
                    cd = plsc.load_gather(conf_v, [di])
                    w_v[b][pl.ds(o, LANES)] = jnp.exp(-jnp.abs(cs - cd))

                start_out(ci, b)

                @pl.when(ci + NBUF < trips)
                def _():
                    start_in(ci + NBUF, b)
            return carry

        assert trips % NBUF == 0
        lax.fori_loop(0, trips // NBUF, outer, 0)
        for b in range(NBUF):
            wait_out(trips - NBUF + b, b)

    return k


def kernel(edge_index, confidences, num_nodes):
    del num_nodes
    n_edges = edge_index.shape[1]
    w = _make_sc_kernel(n_edges, confidences.shape[0])(edge_index, confidences)
    return (edge_index, w)


# speedup vs baseline: 1100.1316x; 1.1314x over previous
import functools

import jax
import jax.numpy as jnp
from jax import lax
from jax.experimental import pallas as pl
from jax.experimental.pallas import tpu as pltpu
from jax.experimental.pallas import tpu_sc as plsc

NUM_CORES = 2
NUM_SUBCORES = 16
LANES = 16
NW = NUM_CORES * NUM_SUBCORES
NBUF = 4
DIST = 2
BLK = 128
CHUNK = 2048


@functools.lru_cache(maxsize=None)
def _make_sc_kernel(n_edges: int, n_nodes: int):
    assert n_edges % CHUNK == 0
    n_chunks = n_edges // CHUNK
    trips = -(-n_chunks // (NW * NBUF)) * NBUF
    mesh = plsc.VectorSubcoreMesh(
        core_axis_name="c", subcore_axis_name="s",
        num_cores=NUM_CORES, num_subcores=NUM_SUBCORES)

    @functools.partial(
        pl.kernel,
        mesh=mesh,
        out_type=(jax.ShapeDtypeStruct((2, n_edges), jnp.int32),
                  jax.ShapeDtypeStruct((n_edges,), jnp.float32)),
        scratch_types=[
            pltpu.VMEM((n_nodes,), jnp.float32),
            [pltpu.VMEM((2, CHUNK), jnp.int32)] * NBUF,
            [pltpu.VMEM((CHUNK,), jnp.float32)] * NBUF,
            [pltpu.SemaphoreType.DMA] * NBUF,
            [pltpu.SemaphoreType.DMA] * NBUF,
            [pltpu.SemaphoreType.DMA] * NBUF,
        ],
        compiler_params=pltpu.CompilerParams(needs_layout_passes=False),
    )
    def k(ei_hbm, conf_hbm, ei_out, w_out, conf_v, ei_v, w_v, sin, sow, soe):
        wid = lax.axis_index("s") * NUM_CORES + lax.axis_index("c")
        pltpu.sync_copy(conf_hbm, conf_v)

        def cbase_of(ci):
            return jnp.minimum(wid + ci * NW, n_chunks - 1) * CHUNK

        def start_in(ci, b):
            ebase = cbase_of(ci)
            pltpu.async_copy(ei_hbm.at[:, pl.ds(ebase, CHUNK)],
                             ei_v[b], sin[b])

        def wait_in(ci, b):
            ebase = cbase_of(ci)
            pltpu.make_async_copy(ei_hbm.at[:, pl.ds(ebase, CHUNK)],
                                  ei_v[b], sin[b]).wait()

        def start_out(ci, b):
            ebase = cbase_of(ci)
            pltpu.async_copy(w_v[b], w_out.at[pl.ds(ebase, CHUNK)], sow[b])
            pltpu.async_copy(ei_v[b], ei_out.at[:, pl.ds(ebase, CHUNK)],
                             soe[b])

        def wait_out(ci, b):
            ebase = cbase_of(ci)
            pltpu.make_async_copy(w_v[b], w_out.at[pl.ds(ebase, CHUNK)],
                                  sow[b]).wait()
            pltpu.make_async_copy(ei_v[b], ei_out.at[:, pl.ds(ebase, CHUNK)],
                                  soe[b]).wait()

        for ci0 in range(DIST):
            start_in(ci0, ci0 % NBUF)

        def outer(g, carry):
            for b in range(NBUF):
                ci = g * NBUF + b
                wait_in(ci, b)

                @plsc.parallel_loop(0, CHUNK, LANES, unroll=8)
                def vec_body(o):
                    si = ei_v[b][0, pl.ds(o, LANES)]
                    di = ei_v[b][1, pl.ds(o, LANES)]
                    cs = plsc.load_gather(conf_v, [si])
                    cd = plsc.load_gather(conf_v, [di])
                    w_v[b][pl.ds(o, LANES)] = jnp.exp(-jnp.abs(cs - cd))

                start_out(ci, b)

                b2 = (b + DIST) % NBUF
                prev = ci - (NBUF - DIST)

                @pl.when(prev >= 0)
                def _():
                    wait_out(prev, b2)

                @pl.when(ci + DIST < trips)
                def _():
                    start_in(ci + DIST, b2)
            return carry

        assert trips % NBUF == 0
        lax.fori_loop(0, trips // NBUF, outer, 0)
        for ci0 in range(trips - (NBUF - DIST), trips):
            wait_out(ci0, ci0 % NBUF)

    return k


def kernel(edge_index, confidences, num_nodes):
    del num_nodes
    n_edges = edge_index.shape[1]
    ei, w = _make_sc_kernel(n_edges, confidences.shape[0])(
        edge_index, confidences)
    return (ei, w)
